# baseline (speedup 1.0000x reference)
import functools

import jax
import jax.numpy as jnp
from jax import lax
from jax.experimental import pallas as pl
from jax.experimental.pallas import tpu as pltpu
from jax.experimental.pallas import tpu_sc as plsc

_N = 10000
_E = 160000
_D = 256
_R = 4
_NC, _NS, _L = 2, 16, 16
_NW = _NC * _NS
_PW = _E // _NW
_C = 128
_NCH = -(-_PW // _C)
_TAIL = _PW - (_NCH - 1) * _C
_PAD = _NCH * _C


def _prescale(h, w_relation):
    bn = 1000

    def body(h_ref, w_ref, out_ref):
        r = pl.program_id(0)
        out_ref[...] = (h_ref[...] * w_ref[pl.ds(r, 1), :])[None].astype(
            jnp.bfloat16)

    return pl.pallas_call(
        body,
        grid=(_R, _N // bn),
        in_specs=[
            pl.BlockSpec((bn, _D), lambda r, i: (i, 0)),
            pl.BlockSpec((_R, _D), lambda r, i: (0, 0)),
        ],
        out_specs=pl.BlockSpec((1, bn, _D), lambda r, i: (r, i, 0)),
        out_shape=jax.ShapeDtypeStruct((_R, _N, _D), jnp.bfloat16),
    )(h, w_relation)


def _cast_bf16(h):
    bn = 1000

    def body(h_ref, out_ref):
        out_ref[...] = h_ref[...].astype(jnp.bfloat16)

    return pl.pallas_call(
        body,
        grid=(_N // bn,),
        in_specs=[pl.BlockSpec((bn, _D), lambda i: (i, 0))],
        out_specs=pl.BlockSpec((bn, _D), lambda i: (i, 0)),
        out_shape=jax.ShapeDtypeStruct((_N, _D), jnp.bfloat16),
    )(h)


_mesh = plsc.VectorSubcoreMesh(
    core_axis_name="c", subcore_axis_name="s", num_cores=_NC, num_subcores=_NS
)


@functools.partial(
    pl.kernel,
    out_type=jax.ShapeDtypeStruct((_E,), jnp.float32),
    mesh=_mesh,
    scratch_types=[
        pltpu.VMEM((_PAD,), jnp.int32),
        pltpu.VMEM((_PAD,), jnp.int32),
        pltpu.VMEM((_PAD,), jnp.int32),
        pltpu.VMEM((_PW,), jnp.int32),
        pltpu.VMEM((2, _C, _D // 2), jnp.int32),
        pltpu.VMEM((2, _C, _D // 2), jnp.int32),
        pltpu.VMEM((_C,), jnp.float32),
        pltpu.SemaphoreType.DMA,
        pltpu.SemaphoreType.DMA,
        pltpu.SemaphoreType.DMA,
        pltpu.SemaphoreType.DMA,
    ],
)
def _distmult_sc(hw_hbm, hb_hbm, src_hbm, dst_hbm, et_hbm, out_hbm,
                 gidx, didx, etv, srcv, s_rows, d_rows, outv,
                 sem_s0, sem_s1, sem_d0, sem_d1):
    sems = ((sem_s0, sem_d0), (sem_s1, sem_d1))
    wid = lax.axis_index("s") * _NC + lax.axis_index("c")
    base = wid * _PW
    lane = lax.iota(jnp.int32, _L)
    zero16 = jnp.zeros((_L,), jnp.int32)

    pltpu.sync_copy(src_hbm.at[pl.ds(base, _PW)], srcv)
    pltpu.sync_copy(dst_hbm.at[pl.ds(base, _PW)], didx.at[pl.ds(0, _PW)])
    pltpu.sync_copy(et_hbm.at[pl.ds(base, _PW)], etv.at[pl.ds(0, _PW)])
    for p in range(_PW, _PAD - _L + 1, _L):
        gidx[pl.ds(p, _L)] = zero16
        didx[pl.ds(p, _L)] = zero16
    gidx[pl.ds(_PAD - _L, _L)] = zero16
    didx[pl.ds(_PAD - _L, _L)] = zero16

    @pl.loop(0, _PW - _L + 1, step=_L)
    def _combine(i):
        sl = pl.ds(pl.multiple_of(i, _L), _L)
        gidx[sl] = etv[sl] * _N + srcv[sl]
    lastsl = pl.ds(_PW - _L, _L)
    gidx[lastsl] = etv[lastsl] * _N + srcv[lastsl]

    def _coff(j):
        return j * _C if isinstance(j, int) else pl.multiple_of(j * _C, _C)

    def _fetch(j, b):
        sl = pl.ds(_coff(j), _C)
        pltpu.async_copy(hw_hbm.at[gidx.at[sl]], s_rows.at[b], sems[b][0])
        pltpu.async_copy(hb_hbm.at[didx.at[sl]], d_rows.at[b], sems[b][1])

    def _consume(j, b, tail=False):
        sl = pl.ds(_coff(j), _C)
        pltpu.make_async_copy(hw_hbm.at[gidx.at[sl]], s_rows.at[b],
                              sems[b][0]).wait()
        pltpu.make_async_copy(hb_hbm.at[didx.at[sl]], d_rows.at[b],
                              sems[b][1]).wait()

        @pl.loop(0, _C // _L)
        def _group(g):
            res = jnp.zeros((_L,), jnp.float32)
            for e2 in range(_L):
                row = g * _L + e2
                acc0 = jnp.zeros((_L,), jnp.float32)
                acc1 = jnp.zeros((_L,), jnp.float32)
                for k in range(_D // (2 * _L)):
                    ksl = pl.ds(k * _L, _L)
                    s32 = s_rows[b, row, ksl]
                    d32 = d_rows[b, row, ksl]
                    s_lo = lax.bitcast_convert_type(s32 << 16, jnp.float32)
                    s_hi = lax.bitcast_convert_type(s32, jnp.float32)
                    d_lo = lax.bitcast_convert_type(d32 << 16, jnp.float32)
                    d_hi = lax.bitcast_convert_type(d32, jnp.float32)
                    acc0 = acc0 + s_lo * d_lo
                    acc1 = acc1 + s_hi * d_hi
                acc = acc0 + acc1
                for sh in (8, 4, 2, 1):
                    perm = jnp.bitwise_xor(lane, sh)
                    acc = acc + jnp.take_along_axis(
                        acc, perm, axis=0, mode="promise_in_bounds")
                res = jnp.where(lane == e2, acc, res)
            outv[pl.ds(pl.multiple_of(g * _L, _L), _L)] = res

        eoff = base + _coff(j)
        if tail:
            pltpu.sync_copy(outv.at[pl.ds(0, _TAIL)],
                            out_hbm.at[pl.ds(eoff, _TAIL)])
        else:
            pltpu.sync_copy(outv, out_hbm.at[pl.ds(eoff, _C)])

    _fetch(0, 0)

    @pl.loop(0, _NCH - 2, step=2)
    def _rounds(t):
        for b in range(2):
            _fetch(t + b + 1, 1 - b)
            _consume(t + b, b)

    _fetch(_NCH - 1, 1)
    _consume(_NCH - 2, 0)
    _consume(_NCH - 1, 1, tail=True)


def kernel(h, edge_index, edge_type, w_relation):
    src = edge_index[0].astype(jnp.int32)
    dst = edge_index[1].astype(jnp.int32)
    et = edge_type.astype(jnp.int32)
    hw = lax.bitcast_convert_type(
        _prescale(h, w_relation).reshape(_R * _N, _D // 2, 2), jnp.int32)
    hb = lax.bitcast_convert_type(
        _cast_bf16(h).reshape(_N, _D // 2, 2), jnp.int32)
    return _distmult_sc(hw, hb, src, dst, et)


# speedup vs baseline: 2.4355x; 2.4355x over previous
import functools

import jax
import jax.numpy as jnp
from jax import lax
from jax.experimental import pallas as pl
from jax.experimental.pallas import tpu as pltpu
from jax.experimental.pallas import tpu_sc as plsc

_N = 10000
_E = 160000
_D = 256
_R = 4
_NC, _NS, _L = 2, 16, 16
_NW = _NC * _NS
_PW = _E // _NW
_C = 128
_NCH = -(-_PW // _C)
_TAIL = _PW - (_NCH - 1) * _C
_PAD = _NCH * _C


def _pack_pair_bf16(lo_f32, hi_f32):
    ulo = lax.bitcast_convert_type(lo_f32, jnp.int32)
    uhi = lax.bitcast_convert_type(hi_f32, jnp.int32)
    rlo = (ulo + 0x7FFF + ((ulo >> 16) & 1)) >> 16
    rhi = (uhi + 0x7FFF + ((uhi >> 16) & 1)) >> 16
    return (rlo & 0xFFFF) | (rhi << 16)


def _prescale(h, w_relation):
    bn = 1000
    hd = _D // 2

    def body(h_ref, w_ref, out_ref):
        r = pl.program_id(0)
        p = h_ref[...] * w_ref[pl.ds(r, 1), :]
        out_ref[...] = _pack_pair_bf16(p[:, :hd], p[:, hd:])[None]

    return pl.pallas_call(
        body,
        grid=(_R, _N // bn),
        in_specs=[
            pl.BlockSpec((bn, _D), lambda r, i: (i, 0)),
            pl.BlockSpec((_R, _D), lambda r, i: (0, 0)),
        ],
        out_specs=pl.BlockSpec((1, bn, hd), lambda r, i: (r, i, 0)),
        out_shape=jax.ShapeDtypeStruct((_R, _N, hd), jnp.int32),
    )(h, w_relation)


def _cast_pack(h):
    bn = 1000
    hd = _D // 2

    def body(h_ref, out_ref):
        x = h_ref[...]
        out_ref[...] = _pack_pair_bf16(x[:, :hd], x[:, hd:])

    return pl.pallas_call(
        body,
        grid=(_N // bn,),
        in_specs=[pl.BlockSpec((bn, _D), lambda i: (i, 0))],
        out_specs=pl.BlockSpec((bn, hd), lambda i: (i, 0)),
        out_shape=jax.ShapeDtypeStruct((_N, hd), jnp.int32),
    )(h)


_mesh = plsc.VectorSubcoreMesh(
    core_axis_name="c", subcore_axis_name="s", num_cores=_NC, num_subcores=_NS
)


@functools.partial(
    pl.kernel,
    out_type=jax.ShapeDtypeStruct((_E,), jnp.float32),
    mesh=_mesh,
    scratch_types=[
        pltpu.VMEM((_PAD,), jnp.int32),
        pltpu.VMEM((_PAD,), jnp.int32),
        pltpu.VMEM((_PAD,), jnp.int32),
        pltpu.VMEM((_PW,), jnp.int32),
        pltpu.VMEM((2, _C, _D // 2), jnp.int32),
        pltpu.VMEM((2, _C, _D // 2), jnp.int32),
        pltpu.VMEM((_C,), jnp.float32),
        pltpu.SemaphoreType.DMA,
        pltpu.SemaphoreType.DMA,
        pltpu.SemaphoreType.DMA,
        pltpu.SemaphoreType.DMA,
    ],
)
def _distmult_sc(hw_hbm, hb_hbm, src_hbm, dst_hbm, et_hbm, out_hbm,
                 gidx, didx, etv, srcv, s_rows, d_rows, outv,
                 sem_s0, sem_s1, sem_d0, sem_d1):
    sems = ((sem_s0, sem_d0), (sem_s1, sem_d1))
    wid = lax.axis_index("s") * _NC + lax.axis_index("c")
    base = wid * _PW
    lane = lax.iota(jnp.int32, _L)
    zero16 = jnp.zeros((_L,), jnp.int32)

    pltpu.sync_copy(src_hbm.at[pl.ds(base, _PW)], srcv)
    pltpu.sync_copy(dst_hbm.at[pl.ds(base, _PW)], didx.at[pl.ds(0, _PW)])
    pltpu.sync_copy(et_hbm.at[pl.ds(base, _PW)], etv.at[pl.ds(0, _PW)])
    for p in range(_PW, _PAD - _L + 1, _L):
        gidx[pl.ds(p, _L)] = zero16
        didx[pl.ds(p, _L)] = zero16
    gidx[pl.ds(_PAD - _L, _L)] = zero16
    didx[pl.ds(_PAD - _L, _L)] = zero16

    @pl.loop(0, _PW - _L + 1, step=_L)
    def _combine(i):
        sl = pl.ds(pl.multiple_of(i, _L), _L)
        gidx[sl] = etv[sl] * _N + srcv[sl]
    lastsl = pl.ds(_PW - _L, _L)
    gidx[lastsl] = etv[lastsl] * _N + srcv[lastsl]

    def _coff(j):
        return j * _C if isinstance(j, int) else pl.multiple_of(j * _C, _C)

    def _fetch(j, b):
        sl = pl.ds(_coff(j), _C)
        pltpu.async_copy(hw_hbm.at[gidx.at[sl]], s_rows.at[b], sems[b][0])
        pltpu.async_copy(hb_hbm.at[didx.at[sl]], d_rows.at[b], sems[b][1])

    def _consume(j, b, tail=False):
        sl = pl.ds(_coff(j), _C)
        pltpu.make_async_copy(hw_hbm.at[gidx.at[sl]], s_rows.at[b],
                              sems[b][0]).wait()
        pltpu.make_async_copy(hb_hbm.at[didx.at[sl]], d_rows.at[b],
                              sems[b][1]).wait()

        @pl.loop(0, _C // _L)
        def _group(g):
            res = jnp.zeros((_L,), jnp.float32)
            for e2 in range(_L):
                row = g * _L + e2
                acc0 = jnp.zeros((_L,), jnp.float32)
                acc1 = jnp.zeros((_L,), jnp.float32)
                for k in range(_D // (2 * _L)):
                    ksl = pl.ds(k * _L, _L)
                    s32 = s_rows[b, row, ksl]
                    d32 = d_rows[b, row, ksl]
                    s_lo = lax.bitcast_convert_type(s32 << 16, jnp.float32)
                    s_hi = lax.bitcast_convert_type(s32, jnp.float32)
                    d_lo = lax.bitcast_convert_type(d32 << 16, jnp.float32)
                    d_hi = lax.bitcast_convert_type(d32, jnp.float32)
                    acc0 = acc0 + s_lo * d_lo
                    acc1 = acc1 + s_hi * d_hi
                acc = acc0 + acc1
                for sh in (8, 4, 2, 1):
                    perm = jnp.bitwise_xor(lane, sh)
                    acc = acc + jnp.take_along_axis(
                        acc, perm, axis=0, mode="promise_in_bounds")
                res = jnp.where(lane == e2, acc, res)
            outv[pl.ds(pl.multiple_of(g * _L, _L), _L)] = res

        eoff = base + _coff(j)
        if tail:
            pltpu.sync_copy(outv.at[pl.ds(0, _TAIL)],
                            out_hbm.at[pl.ds(eoff, _TAIL)])
        else:
            pltpu.sync_copy(outv, out_hbm.at[pl.ds(eoff, _C)])

    _fetch(0, 0)

    @pl.loop(0, _NCH - 2, step=2)
    def _rounds(t):
        for b in range(2):
            _fetch(t + b + 1, 1 - b)
            _consume(t + b, b)

    _fetch(_NCH - 1, 1)
    _consume(_NCH - 2, 0)
    _consume(_NCH - 1, 1, tail=True)


def kernel(h, edge_index, edge_type, w_relation):
    src = edge_index[0].astype(jnp.int32)
    dst = edge_index[1].astype(jnp.int32)
    et = edge_type.astype(jnp.int32)
    hw = _prescale(h, w_relation).reshape(_R * _N, _D // 2)
    hb = _cast_pack(h)
    return _distmult_sc(hw, hb, src, dst, et)


# speedup vs baseline: 4.9157x; 2.0184x over previous
import functools

import jax
import jax.numpy as jnp
from jax import lax
from jax.experimental import pallas as pl
from jax.experimental.pallas import tpu as pltpu
from jax.experimental.pallas import tpu_sc as plsc

_N = 10000
_E = 160000
_D = 256
_HD = _D // 2
_R = 4
_NC, _NS, _L = 2, 16, 16
_NW = _NC * _NS
_PW = _E // _NW
_C = 80
_NCH = -(-_PW // _C)
_TAIL = _PW - (_NCH - 1) * _C
_PAD = _NCH * _C


def _rne16(u):
    return (u + 0x7FFF + ((u >> 16) & 1)) >> 16


def _pack_pair_bf16(lo_f32, hi_f32):
    ulo = lax.bitcast_convert_type(lo_f32, jnp.int32)
    uhi = lax.bitcast_convert_type(hi_f32, jnp.int32)
    return (_rne16(ulo) & 0xFFFF) | (_rne16(uhi) << 16)


def _cast_pack(h):
    bn = 1000

    def body(h_ref, out_ref):
        x = h_ref[...]
        out_ref[...] = _pack_pair_bf16(x[:, :_HD], x[:, _HD:])

    return pl.pallas_call(
        body,
        grid=(_N // bn,),
        in_specs=[pl.BlockSpec((bn, _D), lambda i: (i, 0))],
        out_specs=pl.BlockSpec((bn, _HD), lambda i: (i, 0)),
        out_shape=jax.ShapeDtypeStruct((_N, _HD), jnp.int32),
    )(h)


def _pack_w(w_relation):
    def body(w_ref, out_ref):
        x = w_ref[...]
        out_ref[...] = _pack_pair_bf16(x[:, :_HD], x[:, _HD:])

    return pl.pallas_call(
        body,
        in_specs=[pl.BlockSpec((_R, _D), lambda: (0, 0))],
        out_specs=pl.BlockSpec((_R, _HD), lambda: (0, 0)),
        out_shape=jax.ShapeDtypeStruct((_R, _HD), jnp.int32),
    )(w_relation)


def _pack_idx(src, dst, et):
    rows = _E // 128

    def body(s_ref, d_ref, e_ref, out_ref):
        out_ref[...] = s_ref[...] | (d_ref[...] << 14) | (e_ref[...] << 28)

    spec = pl.BlockSpec((rows, 128), lambda i: (i, 0))
    packed = pl.pallas_call(
        body,
        grid=(_E // 128 // rows,),
        in_specs=[spec, spec, spec],
        out_specs=spec,
        out_shape=jax.ShapeDtypeStruct((_E // 128, 128), jnp.int32),
    )(src.reshape(_E // 128, 128), dst.reshape(_E // 128, 128),
      et.reshape(_E // 128, 128))
    return packed.reshape(_E)


_mesh = plsc.VectorSubcoreMesh(
    core_axis_name="c", subcore_axis_name="s", num_cores=_NC, num_subcores=_NS
)


@functools.partial(
    pl.kernel,
    out_type=jax.ShapeDtypeStruct((_E,), jnp.float32),
    mesh=_mesh,
    compiler_params=pltpu.CompilerParams(needs_layout_passes=False),
    scratch_types=[
        pltpu.VMEM_SHARED((_N, _HD), jnp.int32),
        pltpu.VMEM((_PAD,), jnp.int32),
        pltpu.VMEM((2, _C), jnp.int32),
        pltpu.VMEM((2, _C), jnp.int32),
        pltpu.VMEM((2, _C), jnp.int32),
        pltpu.VMEM((_R * _HD,), jnp.int32),
        pltpu.VMEM((2, _C, _HD), jnp.int32),
        pltpu.VMEM((2, _C, _HD), jnp.int32),
        pltpu.VMEM((_C,), jnp.float32),
        pltpu.SemaphoreType.DMA,
        pltpu.SemaphoreType.DMA,
        pltpu.SemaphoreType.DMA,
        pltpu.SemaphoreType.DMA,
    ],
)
def _distmult_sc(hb_hbm, wp_hbm, pidx_hbm, out_hbm,
                 tab, pidx, sidx, didx, etv, wp, s_rows, d_rows, outv,
                 sem_s0, sem_s1, sem_d0, sem_d1):
    sems = ((sem_s0, sem_d0), (sem_s1, sem_d1))
    sid = lax.axis_index("s")
    wid = sid * _NC + lax.axis_index("c")
    base = wid * _PW
    lane = lax.iota(jnp.int32, _L)
    zero16 = jnp.zeros((_L,), jnp.int32)

    toff = pl.multiple_of(sid * 624, 8)
    pltpu.sync_copy(hb_hbm.at[pl.ds(toff, 624)], tab.at[pl.ds(toff, 624)])

    @pl.when(sid == 0)
    def _tab_rem():
        pltpu.sync_copy(hb_hbm.at[pl.ds(624 * _NS, _N - 624 * _NS)],
                        tab.at[pl.ds(624 * _NS, _N - 624 * _NS)])

    pltpu.sync_copy(wp_hbm, wp)

    pltpu.sync_copy(pidx_hbm.at[pl.ds(base, _PW)], pidx.at[pl.ds(0, _PW)])
    pidx[pl.ds(_PW, _L)] = zero16
    pidx[pl.ds(_PW + _L, _L)] = zero16
    pidx[pl.ds(_PAD - _L, _L)] = zero16

    plsc.subcore_barrier()

    def _coff(j):
        return j * _C if isinstance(j, int) else pl.multiple_of(j * _C, _C)

    def _fetch(j, b):
        off = _coff(j)
        for q in range(_C // _L):
            v = pidx[pl.ds(off + q * _L, _L)]
            sidx[b, pl.ds(q * _L, _L)] = v & 0x3FFF
            didx[b, pl.ds(q * _L, _L)] = (v >> 14) & 0x3FFF
            etv[b, pl.ds(q * _L, _L)] = (v >> 28) & 0x3
        pltpu.async_copy(tab.at[sidx.at[b]], s_rows.at[b], sems[b][0])
        pltpu.async_copy(tab.at[didx.at[b]], d_rows.at[b], sems[b][1])

    def _consume(j, b, tail=False):
        pltpu.make_async_copy(tab.at[sidx.at[b]], s_rows.at[b],
                              sems[b][0]).wait()
        pltpu.make_async_copy(tab.at[didx.at[b]], d_rows.at[b],
                              sems[b][1]).wait()

        @pl.loop(0, _C // _L)
        def _group(g):
            et_vreg = etv[b, pl.ds(pl.multiple_of(g * _L, _L), _L)]
            res = jnp.zeros((_L,), jnp.float32)
            for e2 in range(_L):
                row = g * _L + e2
                et_splat = jnp.take_along_axis(
                    et_vreg, jnp.full((_L,), e2, jnp.int32), axis=0,
                    mode="promise_in_bounds")
                widx = et_splat * _HD + lane
                acc0 = jnp.zeros((_L,), jnp.float32)
                acc1 = jnp.zeros((_L,), jnp.float32)
                for k in range(_HD // _L):
                    ksl = pl.ds(k * _L, _L)
                    s32 = s_rows[b, row, ksl]
                    d32 = d_rows[b, row, ksl]
                    w32 = plsc.load_gather(wp, [widx + (k * _L)])
                    s_lo = lax.bitcast_convert_type(s32 << 16, jnp.float32)
                    s_hi = lax.bitcast_convert_type(s32, jnp.float32)
                    d_lo = lax.bitcast_convert_type(d32 << 16, jnp.float32)
                    d_hi = lax.bitcast_convert_type(d32, jnp.float32)
                    w_lo = lax.bitcast_convert_type(w32 << 16, jnp.float32)
                    w_hi = lax.bitcast_convert_type(w32, jnp.float32)
                    acc0 = acc0 + (s_lo * d_lo) * w_lo
                    acc1 = acc1 + (s_hi * d_hi) * w_hi
                acc = acc0 + acc1
                for sh in (8, 4, 2, 1):
                    perm = jnp.bitwise_xor(lane, sh)
                    acc = acc + jnp.take_along_axis(
                        acc, perm, axis=0, mode="promise_in_bounds")
                res = jnp.where(lane == e2, acc, res)
            outv[pl.ds(pl.multiple_of(g * _L, _L), _L)] = res

        eoff = base + _coff(j)
        if tail:
            pltpu.sync_copy(outv.at[pl.ds(0, _TAIL)],
                            out_hbm.at[pl.ds(eoff, _TAIL)])
        else:
            pltpu.sync_copy(outv, out_hbm.at[pl.ds(eoff, _C)])

    _fetch(0, 0)

    @pl.loop(0, _NCH - 2, step=2)
    def _rounds(t):
        for b in range(2):
            _fetch(t + b + 1, 1 - b)
            _consume(t + b, b)

    _consume(_NCH - 1, 0, tail=True)


def kernel(h, edge_index, edge_type, w_relation):
    src = edge_index[0].astype(jnp.int32)
    dst = edge_index[1].astype(jnp.int32)
    et = edge_type.astype(jnp.int32)
    hb = _cast_pack(h)
    wp = _pack_w(w_relation).reshape(_R * _HD)
    pidx = _pack_idx(src, dst, et)
    return _distmult_sc(hb, wp, pidx)
